# Initial kernel scaffold; baseline (speedup 1.0000x reference)
import jax
import jax.numpy as jnp
from jax.experimental import pallas as pl


def kernel(x, W_in1_1, b_in1_1, W_in1_2, b_in1_2, W_out1_1, b_out1_1, W_out1_2, b_out1_2, W_in2_1, b_in2_1, W_in2_2, b_in2_2, W_out2_1, b_out2_1, W_out2_2, b_out2_2, W_fc, b_fc):
    raise NotImplementedError("write your pallas kernel here")


# speedup vs baseline: 11.9811x; 11.9811x over previous
import functools
import math

import jax
import jax.numpy as jnp
from jax import lax
from jax.experimental import pallas as pl
from jax.experimental.pallas import tpu as pltpu

B, N, C, K = 32, 256, 192, 9

_INV_SQRT2 = 1.0 / math.sqrt(2.0)


def _gelu(x):
    return 0.5 * x * (1.0 + lax.erf(x * _INV_SQRT2))


def _mm(a, b_t):
    return lax.dot_general(a, b_t, (((1,), (1,)), ((), ())),
                           preferred_element_type=jnp.float32)


def _vig_body(x_ref, w11_ref, b11_ref, w12_ref, b12_ref,
              wo11_ref, bo11_ref, wo12_ref, bo12_ref,
              wi21_ref, bi21_ref, wi22_ref, bi22_ref,
              wo21_ref, bo21_ref, wo22_ref, bo22_ref,
              wfh_ref, wfa_ref, bfc_ref, out_ref):
    x = x_ref[0]
    sim = _mm(x, x)

    h = _mm(_gelu(_mm(x, w11_ref[...]) + b11_ref[...]), w12_ref[...]) \
        + b12_ref[...]

    iota_m = lax.broadcasted_iota(jnp.int32, (N, N), 1)
    simw = sim
    m_agg = None
    for _ in range(K):
        rowmax = jnp.max(simw, axis=1, keepdims=True)
        idx = jnp.min(jnp.where(simw == rowmax, iota_m, N), axis=1,
                      keepdims=True)
        onehot = iota_m == idx
        pick = lax.dot_general(onehot.astype(jnp.float32), h,
                               (((1,), (0,)), ((), ())),
                               preferred_element_type=jnp.float32)
        m_agg = pick if m_agg is None else jnp.maximum(m_agg, pick)
        simw = jnp.where(onehot, -jnp.inf, simw)

    y = _mm(h, wfh_ref[...]) + _mm(m_agg, wfa_ref[...]) + bfc_ref[...]
    g = _gelu(y)
    h2 = _mm(_gelu(_mm(g, wo11_ref[...]) + bo11_ref[...]), wo12_ref[...]) \
        + bo12_ref[...]
    x1 = h2 + x

    f = _mm(_gelu(_mm(x1, wi21_ref[...]) + bi21_ref[...]), wi22_ref[...]) \
        + bi22_ref[...]
    f = _gelu(f)
    f = _mm(_gelu(_mm(f, wo21_ref[...]) + bo21_ref[...]), wo22_ref[...]) \
        + bo22_ref[...]
    out_ref[0] = f + x1


def _full(shape):
    return pl.BlockSpec(shape, lambda b: (0,) * len(shape))


@jax.jit
def kernel(x, W_in1_1, b_in1_1, W_in1_2, b_in1_2,
           W_out1_1, b_out1_1, W_out1_2, b_out1_2,
           W_in2_1, b_in2_1, W_in2_2, b_in2_2,
           W_out2_1, b_out2_1, W_out2_2, b_out2_2,
           W_fc, b_fc):
    w_h = W_fc[:, 0::2]
    w_a = W_fc[:, 1::2]
    w_hp = w_h - w_a

    def r2(b):
        return b.reshape(1, -1)

    grid_spec = pl.GridSpec(
        grid=(B,),
        in_specs=[
            pl.BlockSpec((1, N, C), lambda b: (b, 0, 0)),
            _full((C, C)), _full((1, C)),
            _full((C, C)), _full((1, C)),
            _full((C, C)), _full((1, C)),
            _full((C, C)), _full((1, C)),
            _full((4 * C, C)), _full((1, 4 * C)),
            _full((C, 4 * C)), _full((1, C)),
            _full((4 * C, C)), _full((1, 4 * C)),
            _full((C, 4 * C)), _full((1, C)),
            _full((C, C)), _full((C, C)), _full((1, C)),
        ],
        out_specs=pl.BlockSpec((1, N, C), lambda b: (b, 0, 0)),
    )
    return pl.pallas_call(
        _vig_body,
        grid_spec=grid_spec,
        out_shape=jax.ShapeDtypeStruct((B, N, C), jnp.float32),
        compiler_params=pltpu.CompilerParams(
            dimension_semantics=("arbitrary",),
        ),
    )(x, W_in1_1, r2(b_in1_1), W_in1_2, r2(b_in1_2),
      W_out1_1, r2(b_out1_1), W_out1_2, r2(b_out1_2),
      W_in2_1, r2(b_in2_1), W_in2_2, r2(b_in2_2),
      W_out2_1, r2(b_out2_1), W_out2_2, r2(b_out2_2),
      w_hp, w_a, r2(b_fc))


# speedup vs baseline: 12.3757x; 1.0329x over previous
import functools
import math

import jax
import jax.numpy as jnp
import numpy as np
from jax import lax
from jax.experimental import pallas as pl
from jax.experimental.pallas import tpu as pltpu
from jax.experimental.pallas import tpu_sc as plsc

B, N, C, K = 32, 256, 192, 9
CPAD = 256
KPAD = 16
NLANES = 16
G = 4

_INV_SQRT2 = 1.0 / math.sqrt(2.0)

_R_HP = np.zeros((2 * C, C), np.float32)
_R_A = np.zeros((2 * C, C), np.float32)
_R_HP[2 * np.arange(C), np.arange(C)] = 1.0
_R_HP[2 * np.arange(C) + 1, np.arange(C)] = -1.0
_R_A[2 * np.arange(C) + 1, np.arange(C)] = 1.0


def _gelu(x):
    return 0.5 * x * (1.0 + lax.erf(x * _INV_SQRT2))


def _mm(a, b_t):
    return lax.dot_general(a, b_t, (((1,), (1,)), ((), ())),
                           preferred_element_type=jnp.float32)


def _graph_body(x_ref, w11_ref, b11_ref, w12_ref, b12_ref,
                h_ref, idx_ref):
    iota_row = lax.broadcasted_iota(jnp.int32, (1, N), 1).astype(jnp.float32)
    for g in range(G):
        x = x_ref[g]
        sim = _mm(x, x)

        h = _mm(_gelu(_mm(x, w11_ref[...]) + b11_ref[...]),
                w12_ref[...]) + b12_ref[...]
        h_ref[pl.ds(g * N, N), :] = jnp.concatenate(
            [h, jnp.zeros((N, CPAD - C), jnp.float32)], axis=1)

        simw = sim
        rows = []
        for _ in range(K):
            rowmax = jnp.max(simw, axis=1, keepdims=True)
            oh = simw == rowmax
            ohf = oh.astype(jnp.float32)
            idx_row = lax.dot_general(iota_row, ohf,
                                      (((1,), (1,)), ((), ())),
                                      preferred_element_type=jnp.float32)
            rows.append(idx_row)
            simw = jnp.where(oh, -jnp.inf, simw)

        idxf = jnp.concatenate(rows, axis=0)
        idxf = jnp.minimum(idxf, float(N - 1))
        base = ((pl.program_id(0) * G + g) * N).astype(jnp.float32)
        idxg = (idxf + base).astype(jnp.int32)
        pad = jnp.zeros((KPAD - K, N), jnp.int32)
        idx_ref[g] = jnp.concatenate([idxg, pad], axis=0)


CH = 16
NCH = N // CH


def _sc_agg_body(h_hbm, idx_hbm, out_hbm, idx_v, bufs_v, ob_v,
                 gsem0, gsem1, osem0, osem1):
    cid = lax.axis_index("c")
    sid = lax.axis_index("s")
    wid = sid * 2 + cid
    gsems = (gsem0, gsem1)
    osems = (osem0, osem1)

    pltpu.sync_copy(idx_hbm.at[wid], idx_v)

    def fire(ch):
        slot = ch % 2
        return [pltpu.async_copy(h_hbm.at[idx_v.at[k, ch]],
                                 bufs_v.at[slot, k], gsems[slot])
                for k in range(K)]

    def compute(ch):
        slot = ch % 2

        def body(r, carry):
            for j in range(C // NLANES):
                sl = pl.ds(j * NLANES, NLANES)
                v = bufs_v[slot, 0, r, sl]
                for k in range(1, K):
                    v = jnp.maximum(v, bufs_v[slot, k, r, sl])
                ob_v[slot, r, sl] = v
            return carry
        lax.fori_loop(0, CH, body, 0)
        return pltpu.async_copy(
            ob_v.at[slot], out_hbm.at[pl.ds(wid * N + ch * CH, CH)],
            osems[slot])

    pending = fire(0)
    out_pending = [None, None]
    for ch in range(NCH):
        nxt = fire(ch + 1) if ch + 1 < NCH else []
        for hd in pending:
            hd.wait()
        if out_pending[ch % 2] is not None:
            out_pending[ch % 2].wait()
        out_pending[ch % 2] = compute(ch)
        pending = nxt
    for hd in out_pending:
        if hd is not None:
            hd.wait()


@functools.partial(
    pl.kernel,
    out_type=jax.ShapeDtypeStruct((B * N, C), jnp.float32),
    mesh=plsc.VectorSubcoreMesh(core_axis_name="c", subcore_axis_name="s"),
    scratch_types=[
        pltpu.VMEM((KPAD, NCH, CH), jnp.int32),
        pltpu.VMEM((2, K, CH, CPAD), jnp.float32),
        pltpu.VMEM((2, CH, C), jnp.float32),
        pltpu.SemaphoreType.DMA,
        pltpu.SemaphoreType.DMA,
        pltpu.SemaphoreType.DMA,
        pltpu.SemaphoreType.DMA,
    ],
)
def _sc_agg(h_hbm, idx_hbm, out_hbm, idx_v, bufs_v, ob_v,
            gsem0, gsem1, osem0, osem1):
    _sc_agg_body(h_hbm, idx_hbm, out_hbm, idx_v, bufs_v, ob_v,
                 gsem0, gsem1, osem0, osem1)


def _tail_body(x_ref, h_ref, m_ref,
               wo11_ref, bo11_ref, wo12_ref, bo12_ref,
               wi21_ref, bi21_ref, wi22_ref, bi22_ref,
               wo21_ref, bo21_ref, wo22_ref, bo22_ref,
               wfh_ref, wfa_ref, bfc_ref, out_ref):
    for g in range(G):
        x = x_ref[g]
        h = h_ref[pl.ds(g * N, N), :C]
        m = m_ref[pl.ds(g * N, N), :]
        y = _mm(h, wfh_ref[...]) + _mm(m, wfa_ref[...]) + bfc_ref[...]
        g_act = _gelu(y)
        h2 = _mm(_gelu(_mm(g_act, wo11_ref[...]) + bo11_ref[...]),
                 wo12_ref[...]) + bo12_ref[...]
        x1 = h2 + x
        f = _mm(_gelu(_mm(x1, wi21_ref[...]) + bi21_ref[...]),
                wi22_ref[...]) + bi22_ref[...]
        f = _gelu(f)
        f = _mm(_gelu(_mm(f, wo21_ref[...]) + bo21_ref[...]),
                wo22_ref[...]) + bo22_ref[...]
        out_ref[g] = f + x1


def _full(shape):
    return pl.BlockSpec(shape, lambda b: (0,) * len(shape))


@jax.jit
def kernel(x, W_in1_1, b_in1_1, W_in1_2, b_in1_2,
           W_out1_1, b_out1_1, W_out1_2, b_out1_2,
           W_in2_1, b_in2_1, W_in2_2, b_in2_2,
           W_out2_1, b_out2_1, W_out2_2, b_out2_2,
           W_fc, b_fc):
    w_hp = W_fc @ _R_HP
    w_a = W_fc @ _R_A

    def r2(b):
        return b.reshape(1, -1)

    h_flat, idxg = pl.pallas_call(
        _graph_body,
        grid=(B // G,),
        in_specs=[
            pl.BlockSpec((G, N, C), lambda b: (b, 0, 0)),
            _full((C, C)), _full((1, C)),
            _full((C, C)), _full((1, C)),
        ],
        out_specs=[
            pl.BlockSpec((G * N, CPAD), lambda b: (b, 0)),
            pl.BlockSpec((G, KPAD, N), lambda b: (b, 0, 0)),
        ],
        out_shape=[
            jax.ShapeDtypeStruct((B * N, CPAD), jnp.float32),
            jax.ShapeDtypeStruct((B, KPAD, N), jnp.int32),
        ],
        compiler_params=pltpu.CompilerParams(
            dimension_semantics=("arbitrary",),
        ),
    )(x, W_in1_1, r2(b_in1_1), W_in1_2, r2(b_in1_2))

    m_flat = _sc_agg(h_flat, idxg.reshape(B, KPAD, NCH, CH))

    return pl.pallas_call(
        _tail_body,
        grid=(B // G,),
        in_specs=[
            pl.BlockSpec((G, N, C), lambda b: (b, 0, 0)),
            pl.BlockSpec((G * N, CPAD), lambda b: (b, 0)),
            pl.BlockSpec((G * N, C), lambda b: (b, 0)),
            _full((C, C)), _full((1, C)),
            _full((C, C)), _full((1, C)),
            _full((4 * C, C)), _full((1, 4 * C)),
            _full((C, 4 * C)), _full((1, C)),
            _full((4 * C, C)), _full((1, 4 * C)),
            _full((C, 4 * C)), _full((1, C)),
            _full((C, C)), _full((C, C)), _full((1, C)),
        ],
        out_specs=pl.BlockSpec((G, N, C), lambda b: (b, 0, 0)),
        out_shape=jax.ShapeDtypeStruct((B, N, C), jnp.float32),
        compiler_params=pltpu.CompilerParams(
            dimension_semantics=("arbitrary",),
        ),
    )(x, h_flat, m_flat,
      W_out1_1, r2(b_out1_1), W_out1_2, r2(b_out1_2),
      W_in2_1, r2(b_in2_1), W_in2_2, r2(b_in2_2),
      W_out2_1, r2(b_out2_1), W_out2_2, r2(b_out2_2),
      w_hp, w_a, r2(b_fc))
